# baseline (speedup 1.0000x reference)
import functools

import numpy as np
import jax
import jax.numpy as jnp
from jax import lax
from jax.experimental import pallas as pl
from jax.experimental.pallas import tpu as pltpu
from jax.experimental.pallas import tpu_sc as plsc

LANES = 16


_PB = 28672


def _tc_contract_body(w_ref, x_ref, o_ref):
    o_ref[...] = lax.dot_general(
        w_ref[...],
        x_ref[...],
        dimension_numbers=(((1,), (1,)), ((), ())),
        preferred_element_type=jnp.float32,
        precision=lax.Precision.HIGHEST,
    )[0]


def _tc_contract(wt, xt):
    m, c = xt.shape
    return pl.pallas_call(
        _tc_contract_body,
        grid=(m // _PB,),
        in_specs=[
            pl.BlockSpec((1, c), lambda j: (0, 0)),
            pl.BlockSpec((_PB, c), lambda j: (j, 0)),
        ],
        out_specs=pl.BlockSpec((_PB,), lambda j: (j,)),
        out_shape=jax.ShapeDtypeStruct((m,), jnp.float32),
    )(wt, xt)


def _make_sc_stencil(batch_n, n, w, pad, chunk):
    L = chunk + 2 * pad
    slots = n // chunk
    npad = n + 2 * pad
    mesh = plsc.VectorSubcoreMesh(core_axis_name="c", subcore_axis_name="s")

    @functools.partial(
        pl.kernel,
        mesh=mesh,
        out_type=jax.ShapeDtypeStruct((batch_n * n,), jnp.float32),
        scratch_types=[
            pltpu.VMEM((L,), jnp.float32),
            pltpu.VMEM((L,), jnp.float32),
            pltpu.VMEM((L,), jnp.float32),
            pltpu.VMEM((L,), jnp.float32),
            pltpu.VMEM((chunk,), jnp.float32),
            pltpu.VMEM((LANES,), jnp.float32),
        ],
    )
    def sc_stencil(field_hbm, dis_hbm, bias_hbm, out_hbm,
                   f_v, d_v, g_v, t_v, o_v, b_v):
        wid = lax.axis_index("s") * 2 + lax.axis_index("c")
        bidx = wid // slots
        base = (wid % slots) * chunk

        pltpu.sync_copy(field_hbm.at[pl.ds(bidx * npad + base, L)], f_v)
        pltpu.sync_copy(dis_hbm.at[pl.ds(base, L)], d_v)
        pltpu.sync_copy(bias_hbm, b_v)
        bias = b_v[...]
        iota = lax.iota(jnp.int32, LANES)

        def g_body(i, _):
            j = i * LANES
            g_v[pl.ds(j, LANES)] = f_v[pl.ds(j, LANES)] * d_v[pl.ds(j, LANES)]
            return _
        lax.fori_loop(0, L // LANES, g_body, 0)

        def t_body(i, _):
            j = 16 + i * LANES
            gm = g_v[pl.ds(j - 1, LANES)]
            gc = g_v[pl.ds(j, LANES)]
            gp = g_v[pl.ds(j + 1, LANES)]
            t_v[pl.ds(j, LANES)] = gm + gc + gp
            return _
        lax.fori_loop(0, (L - 32) // LANES, t_body, 0)

        def s_body(i, _):
            j = pad + i * LANES
            acc = (t_v[pl.ds(j - w, LANES)] + t_v[pl.ds(j, LANES)]
                   + t_v[pl.ds(j + w, LANES)] + g_v[pl.ds(j, LANES)])
            o_v[pl.ds(i * LANES, LANES)] = d_v[pl.ds(j, LANES)] * acc + bias
            return _
        lax.fori_loop(0, chunk // LANES, s_body, 0)

        pltpu.sync_copy(o_v, out_hbm.at[pl.ds(bidx * n + base, chunk)])

    return sc_stencil


@functools.lru_cache(maxsize=None)
def _dis_vector(h, w):
    n = h * w
    offs = np.array([-w - 1, -1, w - 1, -w, 0, w, -w + 1, 1, w + 1])
    d = np.arange(n)
    valid = (d[:, None] - offs[None, :] >= 0) & (d[:, None] - offs[None, :] < n)
    deg = 1 + valid.sum(axis=1)
    return (deg.astype(np.float32)) ** -0.5


def kernel(batch, labels, W, b):
    B, C, H, Wd = batch.shape
    n = H * Wd
    w = Wd

    xt = batch.transpose(0, 2, 3, 1).reshape(B * n, C)
    field = _tc_contract(W.reshape(1, C), xt).reshape(B, n)

    pad = 240
    chunk = n // 8
    dis = _dis_vector(H, Wd)
    dis_pad = jnp.asarray(np.pad(dis, pad))
    field_pad = jnp.pad(field, ((0, 0), (pad, pad)))
    bias = jnp.broadcast_to(b.astype(jnp.float32), (LANES,))

    sc = _make_sc_stencil(B, n, w, pad, chunk)
    out = sc(field_pad.reshape(-1), dis_pad, bias)
    return out.reshape(B, H, Wd)


# speedup vs baseline: 1.0397x; 1.0397x over previous
import functools

import numpy as np
import jax
import jax.numpy as jnp
from jax import lax
from jax.experimental import pallas as pl
from jax.experimental.pallas import tpu as pltpu
from jax.experimental.pallas import tpu_sc as plsc

LANES = 16


_PB = 14336


def _tc_contract_body(w_ref, x_ref, o_ref):
    o_ref[...] = lax.dot_general(
        w_ref[...],
        x_ref[...],
        dimension_numbers=(((1,), (1,)), ((), ())),
        preferred_element_type=jnp.float32,
        precision=lax.Precision.HIGHEST,
    )[0]


def _tc_contract(wt, xt):
    m, c = xt.shape
    return pl.pallas_call(
        _tc_contract_body,
        grid=(m // _PB,),
        in_specs=[
            pl.BlockSpec((1, c), lambda j: (0, 0)),
            pl.BlockSpec((_PB, c), lambda j: (j, 0)),
        ],
        out_specs=pl.BlockSpec((_PB,), lambda j: (j,)),
        out_shape=jax.ShapeDtypeStruct((m,), jnp.float32),
    )(wt, xt)


def _make_sc_stencil(batch_n, n, w, pad, chunk):
    L = chunk + 2 * pad
    slots = n // chunk
    npad = n + 2 * pad
    mesh = plsc.VectorSubcoreMesh(core_axis_name="c", subcore_axis_name="s")

    @functools.partial(
        pl.kernel,
        mesh=mesh,
        out_type=jax.ShapeDtypeStruct((batch_n * n,), jnp.float32),
        scratch_types=[
            pltpu.VMEM((L,), jnp.float32),
            pltpu.VMEM((L,), jnp.float32),
            pltpu.VMEM((L,), jnp.float32),
            pltpu.VMEM((L,), jnp.float32),
            pltpu.VMEM((chunk,), jnp.float32),
            pltpu.VMEM((LANES,), jnp.float32),
        ],
    )
    def sc_stencil(field_hbm, dis_hbm, bias_hbm, out_hbm,
                   f_v, d_v, g_v, t_v, o_v, b_v):
        wid = lax.axis_index("s") * 2 + lax.axis_index("c")
        bidx = wid // slots
        slot = wid % slots
        base = slot * chunk
        zero = jnp.zeros((LANES,), jnp.float32)

        @pl.when(slot == 0)
        def _():
            for i in range(pad // LANES):
                f_v[pl.ds(i * LANES, LANES)] = zero
            pltpu.sync_copy(field_hbm.at[pl.ds(bidx * n, L - pad)],
                            f_v.at[pl.ds(pad, L - pad)])

        @pl.when(slot == slots - 1)
        def _():
            for i in range(pad // LANES):
                f_v[pl.ds(L - pad + i * LANES, LANES)] = zero
            pltpu.sync_copy(field_hbm.at[pl.ds(bidx * n + base - pad, L - pad)],
                            f_v.at[pl.ds(0, L - pad)])

        @pl.when(jnp.logical_and(slot > 0, slot < slots - 1))
        def _():
            pltpu.sync_copy(field_hbm.at[pl.ds(bidx * n + base - pad, L)], f_v)

        pltpu.sync_copy(dis_hbm.at[pl.ds(base, L)], d_v)
        pltpu.sync_copy(bias_hbm, b_v)
        bias = b_v[...]
        iota = lax.iota(jnp.int32, LANES)

        def g_body(i, _):
            j = i * LANES
            g_v[pl.ds(j, LANES)] = f_v[pl.ds(j, LANES)] * d_v[pl.ds(j, LANES)]
            return _
        lax.fori_loop(0, L // LANES, g_body, 0)

        def t_body(i, _):
            j = 16 + i * LANES
            gm = g_v[pl.ds(j - 1, LANES)]
            gc = g_v[pl.ds(j, LANES)]
            gp = g_v[pl.ds(j + 1, LANES)]
            t_v[pl.ds(j, LANES)] = gm + gc + gp
            return _
        lax.fori_loop(0, (L - 32) // LANES, t_body, 0)

        def s_body(i, _):
            j = pad + i * LANES
            acc = (t_v[pl.ds(j - w, LANES)] + t_v[pl.ds(j, LANES)]
                   + t_v[pl.ds(j + w, LANES)] + g_v[pl.ds(j, LANES)])
            o_v[pl.ds(i * LANES, LANES)] = d_v[pl.ds(j, LANES)] * acc + bias
            return _
        lax.fori_loop(0, chunk // LANES, s_body, 0)

        pltpu.sync_copy(o_v, out_hbm.at[pl.ds(bidx * n + base, chunk)])

    return sc_stencil


@functools.lru_cache(maxsize=None)
def _dis_vector(h, w):
    n = h * w
    offs = np.array([-w - 1, -1, w - 1, -w, 0, w, -w + 1, 1, w + 1])
    d = np.arange(n)
    valid = (d[:, None] - offs[None, :] >= 0) & (d[:, None] - offs[None, :] < n)
    deg = 1 + valid.sum(axis=1)
    return (deg.astype(np.float32)) ** -0.5


def kernel(batch, labels, W, b):
    B, C, H, Wd = batch.shape
    n = H * Wd
    w = Wd

    xt = batch.transpose(0, 2, 3, 1).reshape(B * n, C)
    field = _tc_contract(W.reshape(1, C), xt)

    pad = 240
    chunk = n // 8
    dis = _dis_vector(H, Wd)
    dis_pad = jnp.asarray(np.pad(dis, pad))
    bias = jnp.broadcast_to(b.astype(jnp.float32), (LANES,))

    sc = _make_sc_stencil(B, n, w, pad, chunk)
    out = sc(field, dis_pad, bias)
    return out.reshape(B, H, Wd)
